# Initial kernel scaffold; baseline (speedup 1.0000x reference)
import jax
import jax.numpy as jnp
from jax.experimental import pallas as pl


def kernel(event_type_ids, src_ids, src_mask, dst_ids, dst_mask, event_edge_ids, event_embeddings, event_timestamps, node_ids, edge_ids, edge_index, edge_timestamps, memory, last_update, node_features, edge_features, event_type_emb_w, time_w, time_b, gru_W_ih, gru_W_hh, gru_b_ih, gru_b_hh, Wq, bq, Wk, bk, Wv, bv, We, Wskip, bskip):
    raise NotImplementedError("write your pallas kernel here")


# speedup vs baseline: 3.6177x; 3.6177x over previous
import functools
import math

import jax
import jax.numpy as jnp
from jax.experimental import pallas as pl
from jax.experimental.pallas import tpu as pltpu
from jax.experimental.pallas import tpu_sc as plsc

_B, _S = 4, 256
_N_NODE, _N_EDGE = 512, 1024
_ETYPE_DIM, _MEM_DIM, _TIME_DIM, _EVT_DIM, _OUT_DIM = 8, 32, 16, 32, 32
_MSG_DIM = _ETYPE_DIM + 2 * _MEM_DIM + _TIME_DIM + _EVT_DIM
_NQ = _B * _N_NODE
_NE = _B * _N_EDGE
_NEV = _B * _S

_NC, _NS = 2, 16
_NW = _NC * _NS

_MEM_CNT = _S + _S + _NQ
_MEM_PT = _MEM_CNT // _NW
_NODEF_PT = _NQ // _NW
_EDGEF_PT = _NE // _NW
_LU_CNT = _S + _NE
_LU_PT = _LU_CNT // _NW

_QB = 512
_EB = 512


def _sc_gather_body(mem_hbm, nodef_hbm, edgef_hbm, lu_hbm,
                    memidx_hbm, nodeidx_hbm, edgeidx_hbm, luidx_hbm,
                    mem_out, nodef_out, edgef_out, lu_out,
                    mi_v, ni_v, ei_v, li_v, mr_v, nr_v, er_v, lr_v, sem):
    wid = jax.lax.axis_index("s") * _NC + jax.lax.axis_index("c")
    b_mem = wid * _MEM_PT
    b_nod = wid * _NODEF_PT
    b_edg = wid * _EDGEF_PT
    b_lu = wid * _LU_PT
    pltpu.sync_copy(memidx_hbm.at[pl.ds(b_mem, _MEM_PT)], mi_v)
    pltpu.sync_copy(nodeidx_hbm.at[pl.ds(b_nod, _NODEF_PT)], ni_v)
    pltpu.sync_copy(edgeidx_hbm.at[pl.ds(b_edg, _EDGEF_PT)], ei_v)
    pltpu.sync_copy(luidx_hbm.at[pl.ds(b_lu, _LU_PT)], li_v)
    c0 = pltpu.async_copy(mem_hbm.at[mi_v], mr_v, sem)
    c1 = pltpu.async_copy(nodef_hbm.at[ni_v], nr_v, sem)
    c2 = pltpu.async_copy(edgef_hbm.at[ei_v], er_v, sem)
    c3 = pltpu.async_copy(lu_hbm.at[li_v], lr_v, sem)
    c0.wait()
    c1.wait()
    c2.wait()
    c3.wait()
    pltpu.sync_copy(mr_v, mem_out.at[pl.ds(b_mem, _MEM_PT)])
    pltpu.sync_copy(nr_v, nodef_out.at[pl.ds(b_nod, _NODEF_PT)])
    pltpu.sync_copy(er_v, edgef_out.at[pl.ds(b_edg, _EDGEF_PT)])
    pltpu.sync_copy(lr_v, lu_out.at[pl.ds(b_lu, _LU_PT)])


def _sc_gather(memory, node_features, edge_features, last_update,
               mem_idx, node_idx, edge_idx, lu_idx):
    return pl.kernel(
        _sc_gather_body,
        out_type=(
            jax.ShapeDtypeStruct((_MEM_CNT, _MEM_DIM), jnp.float32),
            jax.ShapeDtypeStruct((_NQ, _EVT_DIM), jnp.float32),
            jax.ShapeDtypeStruct((_NE, _EVT_DIM), jnp.float32),
            jax.ShapeDtypeStruct((_LU_CNT,), jnp.float32),
        ),
        mesh=plsc.VectorSubcoreMesh(core_axis_name="c", subcore_axis_name="s"),
        compiler_params=pltpu.CompilerParams(use_tc_tiling_on_sc=False),
        scratch_types=[
            pltpu.VMEM((_MEM_PT,), jnp.int32),
            pltpu.VMEM((_NODEF_PT,), jnp.int32),
            pltpu.VMEM((_EDGEF_PT,), jnp.int32),
            pltpu.VMEM((_LU_PT,), jnp.int32),
            pltpu.VMEM((_MEM_PT, _MEM_DIM), jnp.float32),
            pltpu.VMEM((_NODEF_PT, _EVT_DIM), jnp.float32),
            pltpu.VMEM((_EDGEF_PT, _EVT_DIM), jnp.float32),
            pltpu.VMEM((_LU_PT,), jnp.float32),
            pltpu.SemaphoreType.DMA,
        ],
    )(memory, node_features, edge_features, last_update,
      mem_idx, node_idx, edge_idx, lu_idx)


def _sigmoid(x):
    return 1.0 / (1.0 + jnp.exp(-x))


def _tanh(x):
    return 1.0 - 2.0 / (jnp.exp(2.0 * x) + 1.0)


def _tc_body(etype0, ts0, srcmask0, dstmask0, evtemb0, lu0, memsrc0, memdst0,
             srcall, dstall, node_col, node_row, memnode, nodef,
             srcv_col, dstv_col, dstv_row, edgets_col, luedge_col, edgef,
             embw, timew, timeb, wih, whh, bih, bhh,
             wq, bq, wk, bk, wv, bv, we, wskip, bskip,
             out_ref):
    f32 = jnp.float32

    et = etype0[...]
    is_node_event = ((et == 3) | (et == 4)).astype(f32)
    is_not_special = 1.0 - ((et == 0) | (et == 1) | (et == 2)).astype(f32)
    t0 = ts0[...]
    dmask = dstmask0[...]
    rel_edge_ts = t0 - lu0[...] * dmask
    t_in = t0 * is_node_event + rel_edge_ts * dmask
    tw = timew[...]
    tb = timeb[...]
    ts_emb = jnp.cos(t_in * tw + tb) * is_not_special
    iota7 = jax.lax.broadcasted_iota(jnp.int32, (1, 7), 1)
    etype_oh = (et == iota7).astype(f32)
    etype_embs = jnp.dot(etype_oh, embw[...], preferred_element_type=f32)
    src_embs = memsrc0[...] * srcmask0[...]
    dst_embs = memdst0[...] * dmask
    evt = evtemb0[...]
    src_msgs = jnp.concatenate(
        [etype_embs, src_embs, dst_embs, ts_emb, evt], axis=1) * is_not_special
    dst_msgs = jnp.concatenate(
        [etype_embs, dst_embs, src_embs, ts_emb, evt], axis=1) * dmask

    sall = srcall[...]
    dall = dstall[...]
    in_src_l, in_dst_l, msrc_l, mdst_l = [], [], [], []
    for qb in range(0, _NQ, _QB):
        nc = node_col[pl.ds(qb, _QB), :]
        es = (nc == sall).astype(f32)
        ed = (nc == dall).astype(f32)
        in_src_l.append(jnp.sum(es, axis=1, keepdims=True))
        in_dst_l.append(jnp.sum(ed, axis=1, keepdims=True))
        es0 = es[:, :_S]
        ed0 = ed[:, :_S]
        cs = jnp.sum(es0, axis=1, keepdims=True)
        cd = jnp.sum(ed0, axis=1, keepdims=True)
        msrc_l.append(jnp.dot(es0, src_msgs, preferred_element_type=f32)
                      / jnp.maximum(cs, 1.0))
        mdst_l.append(jnp.dot(ed0, dst_msgs, preferred_element_type=f32)
                      / jnp.maximum(cd, 1.0))
    in_src = jnp.concatenate(in_src_l, axis=0) > 0.0
    in_dst = jnp.concatenate(in_dst_l, axis=0) > 0.0
    mean_src = jnp.concatenate(msrc_l, axis=0)
    mean_dst = jnp.concatenate(mdst_l, axis=0)

    msg = jnp.where(in_dst, mean_dst, mean_src)
    h = memnode[...]
    gi = jnp.dot(msg, wih[...], preferred_element_type=f32) + bih[...]
    gh = jnp.dot(h, whh[...], preferred_element_type=f32) + bhh[...]
    r = _sigmoid(gi[:, 0:32] + gh[:, 0:32])
    z = _sigmoid(gi[:, 32:64] + gh[:, 32:64])
    n = _tanh(gi[:, 64:96] + r * gh[:, 64:96])
    gru = (1.0 - z) * n + z * h
    new_mem = jnp.where(in_src | in_dst, gru, h)
    x = jnp.concatenate([nodef[...], new_mem], axis=1)

    nrow = node_row[...]
    ncol = node_col[...]
    iota_row_n = jax.lax.broadcasted_iota(jnp.int32, (1, _NQ), 1)
    iota_col_n = jax.lax.broadcasted_iota(jnp.int32, (_NQ, 1), 0)

    def loc_col(vref):
        outs = []
        for eb in range(0, _NE, _EB):
            v = vref[pl.ds(eb, _EB), :]
            cand = jnp.where(v == nrow, iota_row_n, _NQ)
            outs.append(jnp.min(cand, axis=1, keepdims=True))
        l = jnp.concatenate(outs, axis=0)
        return jnp.where(l == _NQ, 0, l)

    def loc_row(vref):
        outs = []
        for eb in range(0, _NE, _EB):
            v = vref[:, pl.ds(eb, _EB)]
            cand = jnp.where(ncol == v, iota_col_n, _NQ)
            outs.append(jnp.min(cand, axis=0, keepdims=True))
        l = jnp.concatenate(outs, axis=1)
        return jnp.where(l == _NQ, 0, l)

    src_loc = loc_col(srcv_col)
    dst_loc = loc_col(dstv_col)
    dst_loc_r = loc_row(dstv_row)

    q = jnp.dot(x, wq[...], preferred_element_type=f32) + bq[...]
    k = jnp.dot(x, wk[...], preferred_element_type=f32) + bk[...]
    v = jnp.dot(x, wv[...], preferred_element_type=f32) + bv[...]
    rel_te = edgets_col[...] - luedge_col[...]
    rte = jnp.cos(rel_te * tw + tb)
    eattr = jnp.concatenate([rte, edgef[...]], axis=1)
    e = jnp.dot(eattr, we[...], preferred_element_type=f32)
    inv = 1.0 / math.sqrt(float(_OUT_DIM))

    logits_parts, ve_parts = [], []
    mx = jnp.full((1, _NQ), -1e30, f32)
    for eb in range(0, _NE, _EB):
        s_eq = (src_loc[eb:eb + _EB] == iota_row_n).astype(f32)
        d_eq = dst_loc[eb:eb + _EB] == iota_row_n
        e_blk = e[eb:eb + _EB]
        ke = jnp.dot(s_eq, k, preferred_element_type=f32) + e_blk
        ve = jnp.dot(s_eq, v, preferred_element_type=f32) + e_blk
        qd = jnp.dot(d_eq.astype(f32), q, preferred_element_type=f32)
        lg = jnp.sum(qd * ke, axis=1, keepdims=True) * inv
        mx = jnp.maximum(mx, jnp.max(jnp.where(d_eq, lg, -1e30),
                                     axis=0, keepdims=True))
        logits_parts.append(lg)
        ve_parts.append(ve)

    den = jnp.zeros((_NQ, 1), f32)
    num = jnp.zeros((_NQ, _OUT_DIM), f32)
    for i, eb in enumerate(range(0, _NE, _EB)):
        d_eq = dst_loc[eb:eb + _EB] == iota_row_n
        mxd = jnp.sum(jnp.where(d_eq, mx, 0.0), axis=1, keepdims=True)
        ex = jnp.exp(logits_parts[i] - mxd)
        d_eq_t = (iota_col_n == dst_loc_r[:, eb:eb + _EB]).astype(f32)
        den = den + jnp.dot(d_eq_t, ex, preferred_element_type=f32)
        num = num + jnp.dot(d_eq_t, ex * ve_parts[i],
                            preferred_element_type=f32)
    out = num / jnp.maximum(den, 1e-16)
    out = out + jnp.dot(x, wskip[...], preferred_element_type=f32) + bskip[...]
    out_ref[...] = out


def _tc_compute(*args):
    return pl.pallas_call(
        _tc_body,
        out_shape=jax.ShapeDtypeStruct((_NQ, _OUT_DIM), jnp.float32),
    )(*args)


def kernel(event_type_ids, src_ids, src_mask, dst_ids, dst_mask,
           event_edge_ids, event_embeddings, event_timestamps, node_ids,
           edge_ids, edge_index, edge_timestamps, memory, last_update,
           node_features, edge_features, event_type_emb_w, time_w, time_b,
           gru_W_ih, gru_W_hh, gru_b_ih, gru_b_hh,
           Wq, bq, Wk, bk, Wv, bv, We, Wskip, bskip):
    i32 = jnp.int32
    node_flat = node_ids.reshape(-1).astype(i32)
    edge_flat = edge_ids.reshape(-1).astype(i32)
    mem_idx = jnp.concatenate(
        [src_ids[0].astype(i32), dst_ids[0].astype(i32), node_flat])
    lu_idx = jnp.concatenate([event_edge_ids[0].astype(i32), edge_flat])

    mem_rows, nodef_rows, edgef_rows, lu_vals = _sc_gather(
        memory, node_features, edge_features, last_update,
        mem_idx, node_flat, edge_flat, lu_idx)

    out = _tc_compute(
        event_type_ids[0][:, None].astype(i32),
        event_timestamps[0][:, None],
        src_mask[0][:, None],
        dst_mask[0][:, None],
        event_embeddings[0],
        lu_vals[:_S][:, None],
        mem_rows[0:_S],
        mem_rows[_S:2 * _S],
        src_ids.reshape(1, -1).astype(i32),
        dst_ids.reshape(1, -1).astype(i32),
        node_flat[:, None],
        node_flat[None, :],
        mem_rows[2 * _S:],
        nodef_rows,
        edge_index[:, 0, :].reshape(-1, 1).astype(i32),
        edge_index[:, 1, :].reshape(-1, 1).astype(i32),
        edge_index[:, 1, :].reshape(1, -1).astype(i32),
        edge_timestamps.reshape(-1, 1),
        lu_vals[_S:][:, None],
        edgef_rows,
        event_type_emb_w,
        time_w[None, :],
        time_b[None, :],
        gru_W_ih.T,
        gru_W_hh.T,
        gru_b_ih[None, :],
        gru_b_hh[None, :],
        Wq, bq[None, :], Wk, bk[None, :], Wv, bv[None, :],
        We, Wskip, bskip[None, :],
    )
    return out.reshape(_B, _N_NODE, _OUT_DIM)
